# Initial kernel scaffold; baseline (speedup 1.0000x reference)
import jax
import jax.numpy as jnp
from jax.experimental import pallas as pl


def kernel(x, lut):
    raise NotImplementedError("write your pallas kernel here")


# speedup vs baseline: 1.3357x; 1.3357x over previous
import functools
import math

import jax
import jax.numpy as jnp
from jax import lax
from jax.experimental import pallas as pl
from jax.experimental.pallas import tpu as pltpu
from jax.experimental.pallas import tpu_sc as plsc

D_MODEL_K = 1024
VOCAB_K = 100000
SCALE_K = math.sqrt(D_MODEL_K)

_info = plsc.get_sparse_core_info()
_NC, _NS, _L = _info.num_cores, _info.num_subcores, _info.num_lanes
_NW = _NC * _NS


def _make_lookup(B: int, D: int):
    assert B % (8 * _NW) == 0 and D % _L == 0
    b_per_w = B // _NW
    CHUNK = 32
    assert b_per_w % CHUNK == 0
    n_chunks = b_per_w // CHUNK
    mesh = plsc.VectorSubcoreMesh(core_axis_name="c", subcore_axis_name="s")

    @functools.partial(
        pl.kernel,
        mesh=mesh,
        out_type=jax.ShapeDtypeStruct((B, D), jnp.float32),
        scratch_types=[
            pltpu.VMEM((b_per_w,), jnp.int32),
            pltpu.VMEM((CHUNK, D), jnp.float32),
            pltpu.VMEM((CHUNK, D), jnp.float32),
            pltpu.SemaphoreType.DMA,
            pltpu.SemaphoreType.DMA,
            pltpu.SemaphoreType.DMA,
            pltpu.SemaphoreType.DMA,
        ],
    )
    def lookup(x_hbm, lut_hbm, out_hbm, idx_v, rows0, rows1, g0, g1, s0, s1):
        wid = lax.axis_index("s") * _NC + lax.axis_index("c")
        base = wid * b_per_w
        pltpu.sync_copy(x_hbm.at[pl.ds(base, b_per_w)], idx_v)

        bufs = (rows0, rows1)
        gsems = (g0, g1)
        ssems = (s0, s1)

        def gather(c):
            return pltpu.async_copy(
                lut_hbm.at[idx_v.at[pl.ds(c * CHUNK, CHUNK)]],
                bufs[c % 2],
                gsems[c % 2],
            )

        def scale(buf):
            def scale_row(i, carry):
                for j in range(D // _L):
                    sl = buf[i, pl.ds(j * _L, _L)]
                    buf[i, pl.ds(j * _L, _L)] = sl * SCALE_K
                return carry

            lax.fori_loop(0, CHUNK, scale_row, 0)

        def store(c):
            return pltpu.async_copy(
                bufs[c % 2],
                out_hbm.at[pl.ds(base + c * CHUNK, CHUNK)],
                ssems[c % 2],
            )

        gathers = [gather(0)]
        stores = []
        for c in range(n_chunks):
            if c + 1 < n_chunks:
                if c >= 1:
                    stores[c - 1].wait()
                gathers.append(gather(c + 1))
            gathers[c].wait()
            scale(bufs[c % 2])
            stores.append(store(c))
        stores[n_chunks - 2].wait()
        stores[n_chunks - 1].wait()

    return lookup


def kernel(x, lut):
    B = x.shape[0] * x.shape[1]
    D = lut.shape[1]
    flat_idx = jnp.reshape(x, (B,)).astype(jnp.int32)
    out = _make_lookup(B, D)(flat_idx, lut)
    return jnp.reshape(out, (*x.shape, D))


# speedup vs baseline: 1.4733x; 1.1030x over previous
import functools
import math

import jax
import jax.numpy as jnp
from jax import lax
from jax.experimental import pallas as pl
from jax.experimental.pallas import tpu as pltpu
from jax.experimental.pallas import tpu_sc as plsc

D_MODEL_K = 1024
VOCAB_K = 100000
SCALE_K = math.sqrt(D_MODEL_K)

_info = plsc.get_sparse_core_info()
_NC, _NS, _L = _info.num_cores, _info.num_subcores, _info.num_lanes
_NW = _NC * _NS


def _make_lookup(B: int, D: int):
    assert B % (8 * _NW) == 0 and D % _L == 0
    b_per_w = B // _NW
    CHUNK = 32
    assert b_per_w % CHUNK == 0
    n_chunks = b_per_w // CHUNK
    mesh = plsc.VectorSubcoreMesh(core_axis_name="c", subcore_axis_name="s")

    @functools.partial(
        pl.kernel,
        mesh=mesh,
        out_type=jax.ShapeDtypeStruct((B, D), jnp.float32),
        scratch_types=[
            pltpu.VMEM((b_per_w,), jnp.int32),
            pltpu.VMEM((CHUNK, D), jnp.float32),
            pltpu.VMEM((CHUNK, D), jnp.float32),
            pltpu.VMEM((CHUNK, D), jnp.float32),
            pltpu.SemaphoreType.DMA,
            pltpu.SemaphoreType.DMA,
            pltpu.SemaphoreType.DMA,
            pltpu.SemaphoreType.DMA,
            pltpu.SemaphoreType.DMA,
            pltpu.SemaphoreType.DMA,
        ],
    )
    def lookup(
        x_hbm, lut_hbm, out_hbm, idx_v, rows0, rows1, rows2, g0, g1, g2, s0, s1, s2
    ):
        wid = lax.axis_index("s") * _NC + lax.axis_index("c")
        base = wid * b_per_w
        pltpu.sync_copy(x_hbm.at[pl.ds(base, b_per_w)], idx_v)

        NBUF = 3
        bufs = (rows0, rows1, rows2)
        gsems = (g0, g1, g2)
        ssems = (s0, s1, s2)

        def gather(c):
            return pltpu.async_copy(
                lut_hbm.at[idx_v.at[pl.ds(c * CHUNK, CHUNK)]],
                bufs[c % NBUF],
                gsems[c % NBUF],
            )

        def scale(buf):
            def scale_row(i, carry):
                for j in range(D // _L):
                    sl = buf[i, pl.ds(j * _L, _L)]
                    buf[i, pl.ds(j * _L, _L)] = sl * SCALE_K
                return carry

            lax.fori_loop(0, CHUNK, scale_row, 0)

        def store(c):
            return pltpu.async_copy(
                bufs[c % NBUF],
                out_hbm.at[pl.ds(base + c * CHUNK, CHUNK)],
                ssems[c % NBUF],
            )

        gathers = [gather(0)]
        stores = []
        for c in range(n_chunks):
            if c + 1 < n_chunks:
                if c >= NBUF - 1:
                    stores[c + 1 - NBUF].wait()
                gathers.append(gather(c + 1))
            gathers[c].wait()
            scale(bufs[c % NBUF])
            stores.append(store(c))
        for c in range(n_chunks - NBUF, n_chunks):
            stores[c].wait()

    return lookup


def kernel(x, lut):
    B = x.shape[0] * x.shape[1]
    D = lut.shape[1]
    flat_idx = jnp.reshape(x, (B,)).astype(jnp.int32)
    out = _make_lookup(B, D)(flat_idx, lut)
    return jnp.reshape(out, (*x.shape, D))
